# baseline (speedup 1.0000x reference)
import functools

import jax
import jax.numpy as jnp
from jax import lax
from jax.experimental import pallas as pl
from jax.experimental.pallas import tpu as pltpu
from jax.experimental.pallas import tpu_sc as plsc

_IN_CH = 128
_OUT_CH = 128
_BATCH = 4096
_LANE = 128

_NC = 2
_NS = 16
_NW = _NC * _NS
_BPW = _BATCH // _NW
_R = 2
_NGROUP = _IN_CH // _R


@jax.jit
def _lookup(table, idx):
    mesh = plsc.VectorSubcoreMesh(core_axis_name="c", subcore_axis_name="s")

    @functools.partial(
        pl.kernel,
        mesh=mesh,
        out_type=(
            jax.ShapeDtypeStruct((_BATCH, _OUT_CH, _IN_CH), jnp.float32),
            jax.ShapeDtypeStruct((_BATCH, 1, _OUT_CH), jnp.float32),
        ),
        scratch_types=[
            pltpu.VMEM((_BPW,), jnp.int32),
            pltpu.VMEM((_R, _BPW, _LANE), jnp.float32),
            pltpu.SemaphoreType.DMA,
            pltpu.SemaphoreType.DMA,
            pltpu.SemaphoreType.DMA,
            pltpu.SemaphoreType.DMA,
        ],
    )
    def k(table_hbm, idx_hbm, w_hbm, b_hbm, idx_v, rows_v,
          sg0, sg1, so0, so1):
        sg = (sg0, sg1)
        so = (so0, so1)
        wid = lax.axis_index("s") * _NC + lax.axis_index("c")
        base = wid * _BPW
        pltpu.sync_copy(idx_hbm.at[wid], idx_v)

        def gather_block(j, slot):
            pltpu.async_copy(
                table_hbm.at[idx_v, pl.ds(j * _LANE, _LANE)],
                rows_v.at[slot], sg[slot])

        for b in range(_R):
            gather_block(b, b)

        def body(g, carry):
            outs = []
            for b in range(_R):
                j = g * _R + b
                pltpu.make_async_copy(
                    table_hbm.at[pl.ds(0, _BPW), pl.ds(0, _LANE)],
                    rows_v.at[b], sg[b]).wait()
                outs.append(pltpu.async_copy(
                    rows_v.at[b], w_hbm.at[pl.ds(base, _BPW), j], so[b]))
            for b in range(_R):
                outs[b].wait()

                @pl.when(g < _NGROUP - 1)
                def _():
                    gather_block((g + 1) * _R + b, b)

            return carry

        lax.fori_loop(0, _NGROUP, body, 0)

        pltpu.async_copy(
            table_hbm.at[idx_v, pl.ds(_IN_CH * _LANE, _LANE)],
            rows_v.at[0], sg[0]).wait()
        pltpu.sync_copy(rows_v.at[0], b_hbm.at[pl.ds(base, _BPW), 0])

    return k(table, idx)


def kernel(table, obj_idx):
    idx = obj_idx.astype(jnp.int32).reshape(_NW, _BPW)
    return _lookup(table, idx)


# speedup vs baseline: 1.0533x; 1.0533x over previous
import functools

import jax
import jax.numpy as jnp
from jax import lax
from jax.experimental import pallas as pl
from jax.experimental.pallas import tpu as pltpu
from jax.experimental.pallas import tpu_sc as plsc

_IN_CH = 128
_OUT_CH = 128
_BATCH = 4096
_LANE = 128

_NC = 2
_NS = 16
_NW = _NC * _NS
_BPW = _BATCH // _NW
_R = 4
_NGROUP = _IN_CH // _R


@jax.jit
def _lookup(table, idx):
    mesh = plsc.VectorSubcoreMesh(core_axis_name="c", subcore_axis_name="s")

    @functools.partial(
        pl.kernel,
        mesh=mesh,
        out_type=(
            jax.ShapeDtypeStruct((_BATCH, _OUT_CH, _IN_CH), jnp.float32),
            jax.ShapeDtypeStruct((_BATCH, 1, _OUT_CH), jnp.float32),
        ),
        scratch_types=[
            pltpu.VMEM((_BPW,), jnp.int32),
            pltpu.VMEM((_R, _BPW, _LANE), jnp.float32),
            pltpu.SemaphoreType.DMA,
            pltpu.SemaphoreType.DMA,
            pltpu.SemaphoreType.DMA,
            pltpu.SemaphoreType.DMA,
            pltpu.SemaphoreType.DMA,
            pltpu.SemaphoreType.DMA,
            pltpu.SemaphoreType.DMA,
            pltpu.SemaphoreType.DMA,
        ],
    )
    def k(table_hbm, idx_hbm, w_hbm, b_hbm, idx_v, rows_v,
          sg0, sg1, sg2, sg3, so0, so1, so2, so3):
        sg = (sg0, sg1, sg2, sg3)
        so = (so0, so1, so2, so3)
        wid = lax.axis_index("s") * _NC + lax.axis_index("c")
        base = wid * _BPW
        pltpu.sync_copy(idx_hbm.at[wid], idx_v)

        def gather_block(j, slot):
            pltpu.async_copy(
                table_hbm.at[idx_v, pl.ds(j * _LANE, _LANE)],
                rows_v.at[slot], sg[slot])

        for b in range(_R):
            gather_block(b, b)

        def body(g, carry):
            outs = []
            for b in range(_R):
                j = g * _R + b
                pltpu.make_async_copy(
                    table_hbm.at[idx_v, pl.ds(0, _LANE)],
                    rows_v.at[b], sg[b]).wait()
                outs.append(pltpu.async_copy(
                    rows_v.at[b], w_hbm.at[pl.ds(base, _BPW), j], so[b]))
            for b in range(_R):
                outs[b].wait()

                @pl.when(g < _NGROUP - 1)
                def _():
                    gather_block((g + 1) * _R + b, b)

            return carry

        lax.fori_loop(0, _NGROUP, body, 0)

        pltpu.async_copy(
            table_hbm.at[idx_v, pl.ds(_IN_CH * _LANE, _LANE)],
            rows_v.at[0], sg[0]).wait()
        pltpu.sync_copy(rows_v.at[0], b_hbm.at[pl.ds(base, _BPW), 0])

    return k(table, idx)


def kernel(table, obj_idx):
    idx = obj_idx.astype(jnp.int32).reshape(_NW, _BPW)
    return _lookup(table, idx)
